# baseline (speedup 1.0000x reference)
import functools

import jax
import jax.numpy as jnp
from jax import lax
from jax.experimental import pallas as pl
from jax.experimental.pallas import tpu as pltpu
from jax.experimental.pallas import tpu_sc as plsc

BATCH = 16384
SEQ = 200
TABLE = 1000000
NUM_CORES = 2
NUM_SUBCORES = 16
NUM_WORKERS = NUM_CORES * NUM_SUBCORES
ROWS_PER_WORKER = BATCH // NUM_WORKERS
CHUNK_ROWS = 32
CHUNK = CHUNK_ROWS * SEQ
NUM_CHUNKS = ROWS_PER_WORKER // CHUNK_ROWS
LANES = 16
COL_STARTS = tuple(j * LANES for j in range(12)) + (SEQ - LANES,)
FULL_COL_STARTS = tuple(j * LANES for j in range(12))
TAIL_START = SEQ - LANES

STAGE = 4000
NUM_STAGE = TABLE // STAGE
STAGE_ROUNDS = -(-NUM_STAGE // NUM_SUBCORES)


def _sc_gather_exp(x_hbm, table_hbm, out_hbm, tab_sh,
                   ixd0, ixd1, ixf0, ixf1, vlf0, vlf1,
                   ti0, ti1, to0, to1, si0, si1, sg0, sg1, so0, so1):
    cid = lax.axis_index("c")
    sid = lax.axis_index("s")
    wid = sid * NUM_CORES + cid
    row_base = wid * ROWS_PER_WORKER

    ixd = (ixd0, ixd1)
    ixf = (ixf0, ixf1)
    vlf = (vlf0, vlf1)
    tis = (ti0, ti1)
    tos = (to0, to1)
    sis = (si0, si1)
    sgs = (sg0, sg1)
    sos = (so0, so1)

    def idx_copy(ch, b):
        return pltpu.make_async_copy(
            x_hbm.at[pl.ds(row_base + ch * CHUNK_ROWS, CHUNK_ROWS), :],
            ixd[b], sis[b])

    idx_copy(0, 0).start()
    idx_copy(1, 1).start()

    def stage_chunk(t):
        return sid + t * NUM_SUBCORES

    def stage_in(t):
        b = t % 2
        return pltpu.make_async_copy(
            table_hbm.at[pl.ds(stage_chunk(t) * STAGE, STAGE)],
            vlf[b].at[pl.ds(0, STAGE)], tis[b])

    def stage_out(t):
        b = t % 2
        return pltpu.make_async_copy(
            vlf[b].at[pl.ds(0, STAGE)],
            tab_sh.at[pl.ds(stage_chunk(t) * STAGE, STAGE)], tos[b])

    @pl.when(stage_chunk(0) < NUM_STAGE)
    def _():
        stage_in(0).start()

    for t in range(STAGE_ROUNDS):
        @pl.when(stage_chunk(t) < NUM_STAGE)
        def _():
            if t >= 1:
                stage_out(t - 1).wait()
            if t + 1 < STAGE_ROUNDS:
                @pl.when(stage_chunk(t + 1) < NUM_STAGE)
                def _():
                    stage_in(t + 1).start()
            stage_in(t).wait()
            stage_out(t).start()

    last_valid = stage_chunk(STAGE_ROUNDS - 1) < NUM_STAGE

    @pl.when(last_valid)
    def _():
        stage_out(STAGE_ROUNDS - 1).wait()

    @pl.when(jnp.logical_not(last_valid))
    def _():
        stage_out(STAGE_ROUNDS - 2).wait()

    plsc.subcore_barrier()

    def repack_in(b):
        def body(r, carry):
            for c in COL_STARTS:
                ixf[b][pl.ds(r * SEQ + c, LANES)] = ixd[b][r, pl.ds(c, LANES)]
            return carry

        lax.fori_loop(0, CHUNK_ROWS, body, 0, unroll=2)

    def repack_out_exp(b):
        def body(i, carry):
            sl = pl.ds(i * LANES, LANES)
            vlf[b][sl] = jnp.exp(vlf[b][sl])
            return carry

        lax.fori_loop(0, CHUNK // LANES, body, 0, unroll=8)

    def gather(b):
        return pltpu.make_async_copy(tab_sh.at[ixf[b]], vlf[b], sgs[b])

    def out_copy(ch, b):
        return pltpu.make_async_copy(
            vlf[b],
            out_hbm.at[pl.ds((row_base + ch * CHUNK_ROWS) * SEQ, CHUNK)],
            sos[b])

    idx_copy(0, 0).wait()
    repack_in(0)
    gather(0).start()

    NPAIR = NUM_CHUNKS // 2

    def pair_body(k, carry):
        p = 2 * k
        idx_copy(p + 1, 1).wait()
        repack_in(1)

        @pl.when(k < NPAIR - 1)
        def _():
            idx_copy(p + 2, 0).start()

        gather(0).wait()
        gather(1).start()

        @pl.when(k >= 1)
        def _():
            out_copy(p - 2, 0).wait()

        repack_out_exp(0)
        out_copy(p, 0).start()

        @pl.when(k < NPAIR - 1)
        def _():
            idx_copy(p + 2, 0).wait()
            repack_in(0)
            idx_copy(p + 3, 1).start()

        gather(1).wait()

        @pl.when(k < NPAIR - 1)
        def _():
            gather(0).start()

        @pl.when(k >= 1)
        def _():
            out_copy(p - 1, 1).wait()

        repack_out_exp(1)
        out_copy(p + 1, 1).start()
        return carry

    lax.fori_loop(0, NPAIR, pair_body, 0)

    out_copy(NUM_CHUNKS - 2, 0).wait()
    out_copy(NUM_CHUNKS - 1, 1).wait()


@jax.jit
def _run(x, table_flat):
    mesh = plsc.VectorSubcoreMesh(core_axis_name="c", subcore_axis_name="s")
    return pl.kernel(
        _sc_gather_exp,
        out_type=jax.ShapeDtypeStruct((BATCH * SEQ,), jnp.float32),
        mesh=mesh,
        scratch_types=[
            pltpu.VMEM_SHARED((TABLE,), jnp.float32),
            pltpu.VMEM((CHUNK_ROWS, SEQ), jnp.int32),
            pltpu.VMEM((CHUNK_ROWS, SEQ), jnp.int32),
            pltpu.VMEM((CHUNK,), jnp.int32),
            pltpu.VMEM((CHUNK,), jnp.int32),
            pltpu.VMEM((CHUNK,), jnp.float32),
            pltpu.VMEM((CHUNK,), jnp.float32),
        ] + [pltpu.SemaphoreType.DMA] * 10,
    )(x, table_flat)


def kernel(x, table):
    return _run(x, table.reshape(-1)).reshape(BATCH, SEQ)


# speedup vs baseline: 1.0022x; 1.0022x over previous
import functools

import jax
import jax.numpy as jnp
from jax import lax
from jax.experimental import pallas as pl
from jax.experimental.pallas import tpu as pltpu
from jax.experimental.pallas import tpu_sc as plsc

BATCH = 16384
SEQ = 200
TABLE = 1000000
NUM_CORES = 2
NUM_SUBCORES = 16
NUM_WORKERS = NUM_CORES * NUM_SUBCORES
ROWS_PER_WORKER = BATCH // NUM_WORKERS
CHUNK_ROWS = 32
CHUNK = CHUNK_ROWS * SEQ
NUM_CHUNKS = ROWS_PER_WORKER // CHUNK_ROWS
LANES = 16
COL_STARTS = tuple(j * LANES for j in range(12)) + (SEQ - LANES,)

STAGE = 4000
NUM_STAGE = TABLE // STAGE
STAGE_ROUNDS = -(-NUM_STAGE // NUM_SUBCORES)


def _sc_gather_exp(x_hbm, table_hbm, out_hbm, tab_sh,
                   ixd0, ixd1, ixf0, ixf1, vlf0, vlf1,
                   ti0, ti1, to0, to1, si0, si1, sg0, sg1, so0, so1):
    cid = lax.axis_index("c")
    sid = lax.axis_index("s")
    wid = sid * NUM_CORES + cid
    row_base = wid * ROWS_PER_WORKER

    ixd = (ixd0, ixd1)
    ixf = (ixf0, ixf1)
    vlf = (vlf0, vlf1)
    tis = (ti0, ti1)
    tos = (to0, to1)
    sis = (si0, si1)
    sgs = (sg0, sg1)
    sos = (so0, so1)

    def idx_copy(ch, b):
        return pltpu.make_async_copy(
            x_hbm.at[pl.ds(row_base + ch * CHUNK_ROWS, CHUNK_ROWS), :],
            ixd[b], sis[b])

    idx_copy(0, 0).start()
    idx_copy(1, 1).start()

    def stage_chunk(t):
        return sid + t * NUM_SUBCORES

    def stage_in(t):
        b = t % 2
        return pltpu.make_async_copy(
            table_hbm.at[pl.ds(stage_chunk(t) * STAGE, STAGE)],
            vlf[b].at[pl.ds(0, STAGE)], tis[b])

    def stage_out(t):
        b = t % 2
        return pltpu.make_async_copy(
            vlf[b].at[pl.ds(0, STAGE)],
            tab_sh.at[pl.ds(stage_chunk(t) * STAGE, STAGE)], tos[b])

    @pl.when(stage_chunk(0) < NUM_STAGE)
    def _():
        stage_in(0).start()

    for t in range(STAGE_ROUNDS):
        @pl.when(stage_chunk(t) < NUM_STAGE)
        def _():
            if t >= 1:
                stage_out(t - 1).wait()
            if t + 1 < STAGE_ROUNDS:
                @pl.when(stage_chunk(t + 1) < NUM_STAGE)
                def _():
                    stage_in(t + 1).start()
            stage_in(t).wait()
            stage_out(t).start()

    last_valid = stage_chunk(STAGE_ROUNDS - 1) < NUM_STAGE

    @pl.when(last_valid)
    def _():
        stage_out(STAGE_ROUNDS - 1).wait()

    @pl.when(jnp.logical_not(last_valid))
    def _():
        stage_out(STAGE_ROUNDS - 2).wait()

    plsc.subcore_barrier()

    def repack_in(b):
        def body(r, carry):
            for c in COL_STARTS:
                ixf[b][pl.ds(r * SEQ + c, LANES)] = ixd[b][r, pl.ds(c, LANES)]
            return carry

        lax.fori_loop(0, CHUNK_ROWS, body, 0, unroll=2)

    def repack_out_exp(b):
        def body(i, carry):
            sl = pl.ds(i * LANES, LANES)
            vlf[b][sl] = jnp.exp(vlf[b][sl])
            return carry

        lax.fori_loop(0, CHUNK // LANES, body, 0, unroll=8)

    def gather(b):
        return pltpu.make_async_copy(tab_sh.at[ixf[b]], vlf[b], sgs[b])

    def out_copy(ch, b):
        return pltpu.make_async_copy(
            vlf[b],
            out_hbm.at[pl.ds((row_base + ch * CHUNK_ROWS) * SEQ, CHUNK)],
            sos[b])

    idx_copy(0, 0).wait()
    repack_in(0)
    gather(0).start()

    NPAIR = NUM_CHUNKS // 2

    def pair_body(k, carry):
        p = 2 * k
        idx_copy(p + 1, 1).wait()
        repack_in(1)

        @pl.when(k < NPAIR - 1)
        def _():
            idx_copy(p + 2, 0).start()

        gather(0).wait()
        gather(1).start()

        @pl.when(k >= 1)
        def _():
            out_copy(p - 2, 0).wait()

        repack_out_exp(0)
        out_copy(p, 0).start()

        @pl.when(k < NPAIR - 1)
        def _():
            idx_copy(p + 2, 0).wait()
            repack_in(0)
            idx_copy(p + 3, 1).start()

        gather(1).wait()

        @pl.when(k < NPAIR - 1)
        def _():
            gather(0).start()

        @pl.when(k >= 1)
        def _():
            out_copy(p - 1, 1).wait()

        repack_out_exp(1)
        out_copy(p + 1, 1).start()
        return carry

    lax.fori_loop(0, NPAIR, pair_body, 0)

    out_copy(NUM_CHUNKS - 2, 0).wait()
    out_copy(NUM_CHUNKS - 1, 1).wait()


@jax.jit
def _run(x, table_flat):
    mesh = plsc.VectorSubcoreMesh(core_axis_name="c", subcore_axis_name="s")
    return pl.kernel(
        _sc_gather_exp,
        out_type=jax.ShapeDtypeStruct((BATCH * SEQ,), jnp.float32),
        mesh=mesh,
        scratch_types=[
            pltpu.VMEM_SHARED((TABLE,), jnp.float32),
            pltpu.VMEM((CHUNK_ROWS, SEQ), jnp.int32),
            pltpu.VMEM((CHUNK_ROWS, SEQ), jnp.int32),
            pltpu.VMEM((CHUNK,), jnp.int32),
            pltpu.VMEM((CHUNK,), jnp.int32),
            pltpu.VMEM((CHUNK,), jnp.float32),
            pltpu.VMEM((CHUNK,), jnp.float32),
        ] + [pltpu.SemaphoreType.DMA] * 10,
    )(x, table_flat)


def kernel(x, table):
    return _run(x, table.reshape(-1)).reshape(BATCH, SEQ)
